# baseline (speedup 1.0000x reference)
import functools

import jax
import jax.numpy as jnp
from jax import lax
from jax.experimental import pallas as pl
from jax.experimental.pallas import tpu as pltpu
from jax.experimental.pallas import tpu_sc as plsc

NUM_EMB = 1024
EMB_DIM = 64
ROW_BLK = 1024


def _argmin_idx_kernel(x_ref, emb_ref, idx_ref):
    xc = x_ref[0]
    emb = emb_ref[...]
    x = xc.T
    a = jnp.sum(x * x, axis=1, keepdims=True)
    b = jnp.sum(emb * emb, axis=1)
    c = lax.dot_general(x, emb, (((1,), (1,)), ((), ())),
                        preferred_element_type=jnp.float32)
    dist = (a + b[None, :]) - 2.0 * c
    m = jnp.min(dist, axis=1, keepdims=True)
    ii = lax.broadcasted_iota(jnp.int32, dist.shape, 1)
    idx_ref[0, 0] = jnp.min(jnp.where(dist == m, ii, NUM_EMB), axis=1)


def _compute_indices(x_nc_hw, embedding):
    n, _, hw = x_nc_hw.shape
    return pl.pallas_call(
        _argmin_idx_kernel,
        grid=(n,),
        in_specs=[
            pl.BlockSpec((1, EMB_DIM, hw), lambda i: (i, 0, 0)),
            pl.BlockSpec((NUM_EMB, EMB_DIM), lambda i: (0, 0)),
        ],
        out_specs=pl.BlockSpec((1, 1, hw), lambda i: (i, 0, 0)),
        out_shape=jax.ShapeDtypeStruct((n, 1, hw), jnp.int32),
    )(x_nc_hw, embedding)


@functools.lru_cache(maxsize=None)
def _make_sc_gather_nchw(n_img, d, hw):
    info = plsc.get_sparse_core_info()
    nc, ns, lanes = info.num_cores, info.num_subcores, info.num_lanes
    nw = nc * ns
    c_per_w = d // nw
    b = n_img * hw
    k_steps = hw // lanes
    mesh = plsc.VectorSubcoreMesh(core_axis_name="c", subcore_axis_name="s")

    @functools.partial(
        pl.kernel, mesh=mesh,
        compiler_params=pltpu.CompilerParams(use_tc_tiling_on_sc=False,
                                             needs_layout_passes=False),
        out_type=jax.ShapeDtypeStruct((n_img, d, hw), jnp.float32),
        scratch_types=[
            pltpu.VMEM((c_per_w, NUM_EMB), jnp.float32),
            pltpu.VMEM((b,), jnp.int32),
            pltpu.VMEM((n_img, c_per_w, hw), jnp.float32),
        ],
    )
    def gather(embt_hbm, idx_hbm, out_hbm, embt_v, idx_v, out_v):
        wid = lax.axis_index("s") * nc + lax.axis_index("c")
        c0 = wid * c_per_w
        pltpu.sync_copy(embt_hbm.at[pl.ds(c0, c_per_w)], embt_v)
        pltpu.sync_copy(idx_hbm, idx_v)

        def body(k, _):
            off = pl.multiple_of(k * lanes, lanes)
            for ni in range(n_img):
                iv = idx_v[pl.ds(ni * hw + off, lanes)]
                for ci in range(c_per_w):
                    g = plsc.load_gather(embt_v.at[ci], [iv])
                    out_v[ni, ci, pl.ds(off, lanes)] = g
            return _

        lax.fori_loop(0, k_steps, body, None)
        for ni in range(n_img):
            pltpu.sync_copy(out_v.at[ni], out_hbm.at[ni, pl.ds(c0, c_per_w)])

    return gather


def kernel(inputs, embedding):
    n, ch, h, w = inputs.shape
    x_nc_hw = inputs.reshape(n, ch, h * w)
    idx = _compute_indices(x_nc_hw, embedding).reshape(-1)
    embt = embedding.T
    out = _make_sc_gather_nchw(n, ch, h * w)(embt, idx)
    return out.reshape(n, ch, h, w)


# speedup vs baseline: 1.0479x; 1.0479x over previous
import functools

import jax
import jax.numpy as jnp
from jax import lax
from jax.experimental import pallas as pl
from jax.experimental.pallas import tpu as pltpu
from jax.experimental.pallas import tpu_sc as plsc

NUM_EMB = 1024
EMB_DIM = 64
ROW_BLK = 1024


def _argmin_idx_kernel(x_ref, emb_ref, idx_ref):
    xc = x_ref[0]
    emb = emb_ref[...]
    x = xc.T
    a = jnp.sum(x * x, axis=1, keepdims=True)
    b = jnp.sum(emb * emb, axis=1)
    c = lax.dot_general(x, emb, (((1,), (1,)), ((), ())),
                        preferred_element_type=jnp.float32)
    dist = (a + b[None, :]) - 2.0 * c
    m = jnp.min(dist, axis=1, keepdims=True)
    ii = lax.broadcasted_iota(jnp.int32, dist.shape, 1)
    idx_ref[0, 0] = jnp.min(jnp.where(dist == m, ii, NUM_EMB), axis=1)


def _compute_indices(x_nc_hw, embedding):
    n, _, hw = x_nc_hw.shape
    return pl.pallas_call(
        _argmin_idx_kernel,
        grid=(n,),
        in_specs=[
            pl.BlockSpec((1, EMB_DIM, hw), lambda i: (i, 0, 0)),
            pl.BlockSpec((NUM_EMB, EMB_DIM), lambda i: (0, 0)),
        ],
        out_specs=pl.BlockSpec((1, 1, hw), lambda i: (i, 0, 0)),
        out_shape=jax.ShapeDtypeStruct((n, 1, hw), jnp.int32),
    )(x_nc_hw, embedding)


@functools.lru_cache(maxsize=None)
def _make_sc_gather_nchw(n_img, d, hw):
    info = plsc.get_sparse_core_info()
    nc, ns, lanes = info.num_cores, info.num_subcores, info.num_lanes
    nw = nc * ns
    c_per_w = d // nw
    b = n_img * hw
    k_steps = hw // lanes
    mesh = plsc.VectorSubcoreMesh(core_axis_name="c", subcore_axis_name="s")

    @functools.partial(
        pl.kernel, mesh=mesh,
        compiler_params=pltpu.CompilerParams(use_tc_tiling_on_sc=False,
                                             needs_layout_passes=False),
        out_type=jax.ShapeDtypeStruct((n_img, d, hw), jnp.float32),
        scratch_types=[
            pltpu.VMEM((c_per_w, NUM_EMB), jnp.float32),
            pltpu.VMEM((b,), jnp.int32),
            pltpu.VMEM((n_img, c_per_w, hw), jnp.float32),
        ],
    )
    def gather(embt_hbm, idx_hbm, out_hbm, embt_v, idx_v, out_v):
        wid = lax.axis_index("s") * nc + lax.axis_index("c")
        c0 = wid * c_per_w
        pltpu.sync_copy(embt_hbm.at[pl.ds(c0, c_per_w)], embt_v)
        pltpu.sync_copy(idx_hbm, idx_v)

        @plsc.parallel_loop(0, k_steps, unroll=8)
        def body(k):
            off = pl.multiple_of(k * lanes, lanes)
            for ni in range(n_img):
                iv = idx_v[pl.ds(ni * hw + off, lanes)]
                for ci in range(c_per_w):
                    g = plsc.load_gather(embt_v.at[ci], [iv])
                    out_v[ni, ci, pl.ds(off, lanes)] = g
        for ni in range(n_img):
            pltpu.sync_copy(out_v.at[ni], out_hbm.at[ni, pl.ds(c0, c_per_w)])

    return gather


def kernel(inputs, embedding):
    n, ch, h, w = inputs.shape
    x_nc_hw = inputs.reshape(n, ch, h * w)
    idx = _compute_indices(x_nc_hw, embedding).reshape(-1)
    embt = embedding.T
    out = _make_sc_gather_nchw(n, ch, h * w)(embt, idx)
    return out.reshape(n, ch, h, w)
